# baseline (speedup 1.0000x reference)
import functools

import jax
import jax.numpy as jnp
from jax import lax
from jax.experimental import pallas as pl
from jax.experimental.pallas import tpu as pltpu
from jax.experimental.pallas import tpu_sc as plsc

_N0 = 50000
_N1 = 12500
_N2 = 12500
_D = 128
_NCLS = 64
_NEXP = 8

_NC = 2
_NS = 16
_L = 16
_NW = _NC * _NS

_ECH = 64
_NSLOT = 2
_RCH = 128
_ACC_ROWS = 12544
_NCHUNKS = _ACC_ROWS // _RCH

_SC_PARAMS = pltpu.CompilerParams(needs_layout_passes=False)


def _round_up(x, m):
    return (x + m - 1) // m * m


def _tile_chunks(s):
    chunk0 = 6 * s + jnp.minimum(s, 2)
    nchunk = 6 + jnp.where(s < 2, 1, 0)
    return chunk0, nchunk


def _make_sc_aggregate(e_pad, frac0=0.5):
    ntot = e_pad // (_NS * _ECH)
    n0 = int(round(ntot * frac0 / _NSLOT)) * _NSLOT
    n0 = max(_NSLOT, min(n0, ntot - _NSLOT))
    n1 = ntot - n0
    assert n0 % _NSLOT == 0 and n1 % _NSLOT == 0
    f32 = jnp.float32

    @functools.partial(
        pl.kernel,
        out_type=(
            jax.ShapeDtypeStruct((_NC, _ACC_ROWS, _D), f32),
            jax.ShapeDtypeStruct((_NC, _ACC_ROWS, _D), f32),
            jax.ShapeDtypeStruct((_NC, _NS * _ACC_ROWS), f32),
        ),
        mesh=plsc.VectorSubcoreMesh(core_axis_name="c", subcore_axis_name="s"),
        scratch_types=[
            [pltpu.VMEM((_ECH,), jnp.int32) for _ in range(_NSLOT)],
            [pltpu.VMEM((_ECH,), jnp.int32) for _ in range(_NSLOT)],
            [pltpu.VMEM((_ECH, _D), f32) for _ in range(_NSLOT)],
            pltpu.VMEM((_ACC_ROWS,), f32),
            pltpu.VMEM((_RCH,), f32),
            pltpu.VMEM((_RCH,), f32),
            pltpu.VMEM_SHARED((_ACC_ROWS, _D), f32),
            [pltpu.SemaphoreType.DMA for _ in range(_NSLOT)],
            [pltpu.SemaphoreType.DMA for _ in range(_NSLOT)],
        ],
        compiler_params=_SC_PARAMS,
    )
    def agg(table_hbm, src_hbm, dst_hbm, feat_hbm, deg_hbm, dstage,
            sidx, didx, rows, dflat, dmerge, dtmp, acc, sem_i, sem_g):
        c = lax.axis_index("c")
        s = lax.axis_index("s")
        chunk0, nchunk = _tile_chunks(s)
        n = jnp.where(c == 0, n0, n1)
        base = jnp.where(c == 0, s * n0, _NS * n0 + s * n1) * _ECH
        zeros_l = jnp.zeros((_L,), f32)
        ones_l = jnp.ones((_L,), f32)

        def _start_idx(j, b):
            off = base + j * _ECH
            pltpu.async_copy(src_hbm.at[pl.ds(off, _ECH)], sidx[b], sem_i[b])
            pltpu.async_copy(dst_hbm.at[pl.ds(off, _ECH)], didx[b], sem_i[b])

        def _wait_idx(j, b):
            off = base + j * _ECH
            pltpu.make_async_copy(src_hbm.at[pl.ds(off, _ECH)], sidx[b],
                                  sem_i[b]).wait()
            pltpu.make_async_copy(dst_hbm.at[pl.ds(off, _ECH)], didx[b],
                                  sem_i[b]).wait()

        def _start_gather(b):
            pltpu.async_copy(table_hbm.at[sidx[b]], rows[b], sem_g[b])

        def _wait_gather(b):
            pltpu.make_async_copy(table_hbm.at[sidx[b]], rows[b],
                                  sem_g[b]).wait()

        def _zrow(i, carry):
            for jj in range(_D // _L):
                rows[0][i, pl.ds(jj * _L, _L)] = zeros_l
            return carry
        lax.fori_loop(0, _ECH, _zrow, 0)

        def _zflat(i, carry):
            dflat[pl.ds(i * _L, _L)] = zeros_l
            return carry
        lax.fori_loop(0, _ACC_ROWS // _L, _zflat, 0)

        def _zacc(q, carry):
            r = (chunk0 + q) * _RCH
            for h in range(_RCH // _ECH):
                pltpu.sync_copy(rows[0], acc.at[pl.ds(r + h * _ECH, _ECH)])
            return carry
        lax.fori_loop(0, nchunk, _zacc, 0)

        for b in range(_NSLOT):
            _start_idx(b, b)
        _wait_idx(0, 0)
        _start_gather(0)
        plsc.subcore_barrier()

        def _group(g, carry):
            for b in range(_NSLOT):
                j = g * _NSLOT + b
                q = (b + 1) % _NSLOT
                _wait_gather(b)

                @pl.when(j + 1 < n)
                def _():
                    _wait_idx(j + 1, q)
                    _start_gather(q)

                pltpu.sync_copy(rows[b], acc.at[didx[b]], add=True)
                for k in range(_ECH // _L):
                    d16 = didx[b][pl.ds(k * _L, _L)]
                    plsc.addupdate_scatter(dflat, [d16], ones_l)

                @pl.when(j + _NSLOT < n)
                def _():
                    _start_idx(j + _NSLOT, b)
            return carry
        lax.fori_loop(0, n // _NSLOT, _group, 0)

        pltpu.sync_copy(dflat, dstage.at[c, pl.ds(s * _ACC_ROWS, _ACC_ROWS)])
        plsc.subcore_barrier()

        def _out(q, carry):
            r = (chunk0 + q) * _RCH
            pltpu.sync_copy(acc.at[pl.ds(r, _RCH)],
                            feat_hbm.at[c, pl.ds(r, _RCH)])
            pltpu.sync_copy(dstage.at[c, pl.ds(r, _RCH)], dmerge)
            for t in range(1, _NS):
                pltpu.sync_copy(dstage.at[c, pl.ds(t * _ACC_ROWS + r, _RCH)],
                                dtmp)
                for i in range(_RCH // _L):
                    sl = pl.ds(i * _L, _L)
                    dmerge[sl] = dmerge[sl] + dtmp[sl]

            for h in range(_RCH // _ECH):
                def _expand(i, carry2):
                    splat = plsc.load_gather(
                        dmerge, [jnp.full((_L,), h * _ECH + i, jnp.int32)])
                    for jj in range(_D // _L):
                        rows[0][i, pl.ds(jj * _L, _L)] = splat
                    return carry2
                lax.fori_loop(0, _ECH, _expand, 0)
                pltpu.sync_copy(rows[0],
                                deg_hbm.at[c, pl.ds(r + h * _ECH, _ECH)])
            return carry
        lax.fori_loop(0, nchunk, _out, 0)

    return agg


_DOT = dict(precision=lax.Precision.DEFAULT, preferred_element_type=jnp.float32)
_CN = (((1,), (1,)), ((), ()))
_BLK = 1568


def _merged_mean(f_ref, d_ref):
    feat = f_ref[0] + f_ref[1]
    deg = d_ref[0] + d_ref[1]
    return feat / jnp.maximum(deg, 1.0)


def _phase_b_body(x_ref, f_ref, d_ref, ws_ref, wn_ref, b_ref, o_ref):
    neigh = _merged_mean(f_ref, d_ref)
    h = lax.dot_general(x_ref[...], ws_ref[...], _CN, **_DOT)
    h = h + lax.dot_general(neigh, wn_ref[...], _CN, **_DOT)
    o_ref[...] = jnp.maximum(h + b_ref[...], 0.0)


def _phase_d_body(ht_ref, f_ref, d_ref, ws_ref, wn_ref, b_ref, wp_ref,
                  wpp_ref, o_ref):
    neigh = _merged_mean(f_ref, d_ref)
    h2 = lax.dot_general(ht_ref[...], ws_ref[...], _CN, **_DOT)
    h2 = h2 + lax.dot_general(neigh, wn_ref[...], _CN, **_DOT)
    h2 = jnp.maximum(h2 + b_ref[...], 0.0)
    logits = lax.dot_general(h2, wp_ref[...], _CN, **_DOT)
    m = jnp.max(logits, axis=1, keepdims=True)
    eid = lax.broadcasted_iota(jnp.int32, (_BLK, _NEXP), 1)
    cand = jnp.where(logits >= m, eid, _NEXP)
    idx = jnp.min(cand, axis=1, keepdims=True)
    out = jnp.zeros((_BLK, _NCLS), jnp.float32)
    for e in range(_NEXP):
        pe = lax.dot_general(h2, wpp_ref[e * _NCLS:(e + 1) * _NCLS, :],
                             _CN, **_DOT)
        out = out + jnp.where(idx == e, 1.0, 0.0) * pe
    o_ref[...] = out


def kernel(inputs, src_0, dst_0, src_1, dst_1,
           W_self_0, W_neigh_0, b_0, W_self_1, W_neigh_1, b_1,
           W_prompt, W_pp):
    f32 = jnp.float32
    e0p = _round_up(src_0.shape[0], _NW * _ECH * _NSLOT)
    e1p = _round_up(src_1.shape[0], _NW * _ECH * _NSLOT)

    def _pad_edges(src, dst, e_pad):
        pad = e_pad - src.shape[0]
        src = jnp.concatenate([src, jnp.zeros((pad,), jnp.int32)])
        dst = jnp.concatenate([dst, jnp.full((pad,), _N1, jnp.int32)])
        return src, dst

    src0, dst0 = _pad_edges(src_0, dst_0, e0p)
    src1, dst1 = _pad_edges(src_1, dst_1, e1p)

    feat0, deg0, _ = _make_sc_aggregate(e0p, 0.60)(inputs, src0, dst0)

    grid = (_ACC_ROWS // _BLK,)
    _rows = lambda i: (i, 0)
    _pair = lambda i: (0, i, 0)
    _full = lambda i: (0, 0)

    x_dst = jnp.zeros((_ACC_ROWS, _D), f32).at[:_N1].set(inputs[:_N1])
    h1_table = pl.pallas_call(
        _phase_b_body,
        grid=grid,
        in_specs=[
            pl.BlockSpec((_BLK, _D), _rows),
            pl.BlockSpec((_NC, _BLK, _D), _pair),
            pl.BlockSpec((_NC, _BLK, _D), _pair),
            pl.BlockSpec((_D, _D), _full),
            pl.BlockSpec((_D, _D), _full),
            pl.BlockSpec((1, _D), _full),
        ],
        out_specs=pl.BlockSpec((_BLK, _D), _rows),
        out_shape=jax.ShapeDtypeStruct((_ACC_ROWS, _D), f32),
    )(x_dst, feat0, deg0, W_self_0, W_neigh_0, b_0.reshape(1, _D))

    feat1, deg1, _ = _make_sc_aggregate(e1p, 0.75)(h1_table, src1, dst1)

    out_pad = pl.pallas_call(
        _phase_d_body,
        grid=grid,
        in_specs=[
            pl.BlockSpec((_BLK, _D), _rows),
            pl.BlockSpec((_NC, _BLK, _D), _pair),
            pl.BlockSpec((_NC, _BLK, _D), _pair),
            pl.BlockSpec((_D, _D), _full),
            pl.BlockSpec((_D, _D), _full),
            pl.BlockSpec((1, _D), _full),
            pl.BlockSpec((_NEXP, _D), _full),
            pl.BlockSpec((_NEXP * _NCLS, _D), _full),
        ],
        out_specs=pl.BlockSpec((_BLK, _NCLS), _rows),
        out_shape=jax.ShapeDtypeStruct((_ACC_ROWS, _NCLS), f32),
    )(h1_table, feat1, deg1, W_self_1, W_neigh_1, b_1.reshape(1, _D),
      W_prompt, W_pp.reshape(_NEXP * _NCLS, _D))

    return out_pad[:_N2]


# speedup vs baseline: 1.0322x; 1.0322x over previous
import functools

import jax
import jax.numpy as jnp
from jax import lax
from jax.experimental import pallas as pl
from jax.experimental.pallas import tpu as pltpu
from jax.experimental.pallas import tpu_sc as plsc

_N0 = 50000
_N1 = 12500
_N2 = 12500
_D = 128
_NCLS = 64
_NEXP = 8

_NC = 2
_NS = 16
_L = 16
_NW = _NC * _NS

_ECH = 64
_NSLOT = 2
_RCH = 128
_ACC_ROWS = 12544
_NCHUNKS = _ACC_ROWS // _RCH

_SC_PARAMS = pltpu.CompilerParams(needs_layout_passes=False)


def _round_up(x, m):
    return (x + m - 1) // m * m


def _tile_chunks(s):
    chunk0 = 6 * s + jnp.minimum(s, 2)
    nchunk = 6 + jnp.where(s < 2, 1, 0)
    return chunk0, nchunk


def _make_sc_aggregate(e_pad, frac0=0.5):
    ntot = e_pad // (_NS * _ECH)
    n0 = int(round(ntot * frac0 / _NSLOT)) * _NSLOT
    n0 = max(_NSLOT, min(n0, ntot - _NSLOT))
    n1 = ntot - n0
    assert n0 % _NSLOT == 0 and n1 % _NSLOT == 0
    f32 = jnp.float32

    @functools.partial(
        pl.kernel,
        out_type=(
            jax.ShapeDtypeStruct((_NC, _ACC_ROWS, _D), f32),
            jax.ShapeDtypeStruct((_NC, _ACC_ROWS, _D), f32),
            jax.ShapeDtypeStruct((_NC, _NS * _ACC_ROWS), f32),
        ),
        mesh=plsc.VectorSubcoreMesh(core_axis_name="c", subcore_axis_name="s"),
        scratch_types=[
            [pltpu.VMEM((_ECH,), jnp.int32) for _ in range(_NSLOT)],
            [pltpu.VMEM((_ECH,), jnp.int32) for _ in range(_NSLOT)],
            [pltpu.VMEM((_ECH, _D), f32) for _ in range(_NSLOT)],
            pltpu.VMEM((_ACC_ROWS,), f32),
            pltpu.VMEM((_RCH,), f32),
            pltpu.VMEM((_RCH,), f32),
            pltpu.VMEM_SHARED((_ACC_ROWS, _D), f32),
            [pltpu.SemaphoreType.DMA for _ in range(_NSLOT)],
            [pltpu.SemaphoreType.DMA for _ in range(_NSLOT)],
        ],
        compiler_params=_SC_PARAMS,
    )
    def agg(table_hbm, src_hbm, dst_hbm, feat_hbm, deg_hbm, dstage,
            sidx, didx, rows, dflat, dmerge, dtmp, acc, sem_i, sem_g):
        c = lax.axis_index("c")
        s = lax.axis_index("s")
        chunk0, nchunk = _tile_chunks(s)
        n = jnp.where(c == 0, n0, n1)
        base = jnp.where(c == 0, s * n0, _NS * n0 + s * n1) * _ECH
        zeros_l = jnp.zeros((_L,), f32)
        ones_l = jnp.ones((_L,), f32)

        def _start_idx(j, b):
            off = base + j * _ECH
            pltpu.async_copy(src_hbm.at[pl.ds(off, _ECH)], sidx[b], sem_i[b])
            pltpu.async_copy(dst_hbm.at[pl.ds(off, _ECH)], didx[b], sem_i[b])

        def _wait_idx(j, b):
            off = base + j * _ECH
            pltpu.make_async_copy(src_hbm.at[pl.ds(off, _ECH)], sidx[b],
                                  sem_i[b]).wait()
            pltpu.make_async_copy(dst_hbm.at[pl.ds(off, _ECH)], didx[b],
                                  sem_i[b]).wait()

        def _start_gather(b):
            pltpu.async_copy(table_hbm.at[sidx[b]], rows[b], sem_g[b])

        def _wait_gather(b):
            pltpu.make_async_copy(table_hbm.at[sidx[b]], rows[b],
                                  sem_g[b]).wait()

        def _zrow(i, carry):
            for jj in range(_D // _L):
                rows[0][i, pl.ds(jj * _L, _L)] = zeros_l
            return carry
        lax.fori_loop(0, _ECH, _zrow, 0)

        def _zflat(i, carry):
            dflat[pl.ds(i * _L, _L)] = zeros_l
            return carry
        lax.fori_loop(0, _ACC_ROWS // _L, _zflat, 0)

        def _zacc(q, carry):
            r = (chunk0 + q) * _RCH
            for h in range(_RCH // _ECH):
                pltpu.sync_copy(rows[0], acc.at[pl.ds(r + h * _ECH, _ECH)])
            return carry
        lax.fori_loop(0, nchunk, _zacc, 0)

        for b in range(_NSLOT):
            _start_idx(b, b)
        _wait_idx(0, 0)
        _start_gather(0)
        plsc.subcore_barrier()

        def _group(g, carry):
            for b in range(_NSLOT):
                j = g * _NSLOT + b
                q = (b + 1) % _NSLOT
                _wait_gather(b)

                @pl.when(j + 1 < n)
                def _():
                    _wait_idx(j + 1, q)
                    _start_gather(q)

                pltpu.sync_copy(rows[b], acc.at[didx[b]], add=True)
                for k in range(_ECH // _L):
                    d16 = didx[b][pl.ds(k * _L, _L)]
                    plsc.addupdate_scatter(dflat, [d16], ones_l)

                @pl.when(j + _NSLOT < n)
                def _():
                    _start_idx(j + _NSLOT, b)
            return carry
        lax.fori_loop(0, n // _NSLOT, _group, 0)

        pltpu.sync_copy(dflat, dstage.at[c, pl.ds(s * _ACC_ROWS, _ACC_ROWS)])
        plsc.subcore_barrier()

        def _out(q, carry):
            r = (chunk0 + q) * _RCH
            pltpu.sync_copy(acc.at[pl.ds(r, _RCH)],
                            feat_hbm.at[c, pl.ds(r, _RCH)])
            pltpu.sync_copy(dstage.at[c, pl.ds(r, _RCH)], dmerge)
            for t in range(1, _NS):
                pltpu.sync_copy(dstage.at[c, pl.ds(t * _ACC_ROWS + r, _RCH)],
                                dtmp)
                for i in range(_RCH // _L):
                    sl = pl.ds(i * _L, _L)
                    dmerge[sl] = dmerge[sl] + dtmp[sl]

            for h in range(_RCH // _ECH):
                def _expand(i, carry2):
                    splat = plsc.load_gather(
                        dmerge, [jnp.full((_L,), h * _ECH + i, jnp.int32)])
                    for jj in range(_D // _L):
                        rows[0][i, pl.ds(jj * _L, _L)] = splat
                    return carry2
                lax.fori_loop(0, _ECH, _expand, 0)
                pltpu.sync_copy(rows[0],
                                deg_hbm.at[c, pl.ds(r + h * _ECH, _ECH)])
            return carry
        lax.fori_loop(0, nchunk, _out, 0)

    return agg


_DOT = dict(precision=lax.Precision.DEFAULT, preferred_element_type=jnp.float32)
_CN = (((1,), (1,)), ((), ()))
_BLK = 1568


def _merged_mean(f_ref, d_ref):
    feat = f_ref[0] + f_ref[1]
    deg = d_ref[0] + d_ref[1]
    return feat / jnp.maximum(deg, 1.0)


def _phase_b_body(x_ref, f_ref, d_ref, ws_ref, wn_ref, b_ref, o_ref):
    neigh = _merged_mean(f_ref, d_ref)
    h = lax.dot_general(x_ref[...], ws_ref[...], _CN, **_DOT)
    h = h + lax.dot_general(neigh, wn_ref[...], _CN, **_DOT)
    o_ref[...] = jnp.maximum(h + b_ref[...], 0.0)


def _phase_d_body(ht_ref, f_ref, d_ref, ws_ref, wn_ref, b_ref, wp_ref,
                  wpp_ref, o_ref):
    neigh = _merged_mean(f_ref, d_ref)
    h2 = lax.dot_general(ht_ref[...], ws_ref[...], _CN, **_DOT)
    h2 = h2 + lax.dot_general(neigh, wn_ref[...], _CN, **_DOT)
    h2 = jnp.maximum(h2 + b_ref[...], 0.0)
    logits = lax.dot_general(h2, wp_ref[...], _CN, **_DOT)
    m = jnp.max(logits, axis=1, keepdims=True)
    eid = lax.broadcasted_iota(jnp.int32, (_BLK, _NEXP), 1)
    cand = jnp.where(logits >= m, eid, _NEXP)
    idx = jnp.min(cand, axis=1, keepdims=True)
    out = jnp.zeros((_BLK, _NCLS), jnp.float32)
    for e in range(_NEXP):
        pe = lax.dot_general(h2, wpp_ref[e * _NCLS:(e + 1) * _NCLS, :],
                             _CN, **_DOT)
        out = out + jnp.where(idx == e, 1.0, 0.0) * pe
    o_ref[...] = out


def kernel(inputs, src_0, dst_0, src_1, dst_1,
           W_self_0, W_neigh_0, b_0, W_self_1, W_neigh_1, b_1,
           W_prompt, W_pp):
    f32 = jnp.float32
    e0p = _round_up(src_0.shape[0], _NW * _ECH * _NSLOT)
    e1p = _round_up(src_1.shape[0], _NW * _ECH * _NSLOT)

    def _pad_edges(src, dst, e_pad):
        pad = e_pad - src.shape[0]
        src = jnp.concatenate([src, jnp.zeros((pad,), jnp.int32)])
        dst = jnp.concatenate([dst, jnp.full((pad,), _N1, jnp.int32)])
        return src, dst

    src0, dst0 = _pad_edges(src_0, dst_0, e0p)
    src1, dst1 = _pad_edges(src_1, dst_1, e1p)

    feat0, deg0, _ = _make_sc_aggregate(e0p, 0.56)(inputs, src0, dst0)

    grid = (_ACC_ROWS // _BLK,)
    _rows = lambda i: (i, 0)
    _pair = lambda i: (0, i, 0)
    _full = lambda i: (0, 0)

    x_dst = jnp.zeros((_ACC_ROWS, _D), f32).at[:_N1].set(inputs[:_N1])
    h1_table = pl.pallas_call(
        _phase_b_body,
        grid=grid,
        in_specs=[
            pl.BlockSpec((_BLK, _D), _rows),
            pl.BlockSpec((_NC, _BLK, _D), _pair),
            pl.BlockSpec((_NC, _BLK, _D), _pair),
            pl.BlockSpec((_D, _D), _full),
            pl.BlockSpec((_D, _D), _full),
            pl.BlockSpec((1, _D), _full),
        ],
        out_specs=pl.BlockSpec((_BLK, _D), _rows),
        out_shape=jax.ShapeDtypeStruct((_ACC_ROWS, _D), f32),
    )(x_dst, feat0, deg0, W_self_0, W_neigh_0, b_0.reshape(1, _D))

    feat1, deg1, _ = _make_sc_aggregate(e1p, 0.70)(h1_table, src1, dst1)

    out_pad = pl.pallas_call(
        _phase_d_body,
        grid=grid,
        in_specs=[
            pl.BlockSpec((_BLK, _D), _rows),
            pl.BlockSpec((_NC, _BLK, _D), _pair),
            pl.BlockSpec((_NC, _BLK, _D), _pair),
            pl.BlockSpec((_D, _D), _full),
            pl.BlockSpec((_D, _D), _full),
            pl.BlockSpec((1, _D), _full),
            pl.BlockSpec((_NEXP, _D), _full),
            pl.BlockSpec((_NEXP * _NCLS, _D), _full),
        ],
        out_specs=pl.BlockSpec((_BLK, _NCLS), _rows),
        out_shape=jax.ShapeDtypeStruct((_ACC_ROWS, _NCLS), f32),
    )(h1_table, feat1, deg1, W_self_1, W_neigh_1, b_1.reshape(1, _D),
      W_prompt, W_pp.reshape(_NEXP * _NCLS, _D))

    return out_pad[:_N2]
